# baseline (speedup 1.0000x reference)
import numpy as np
import jax
import jax.numpy as jnp
from jax.experimental import pallas as pl
from jax.experimental.pallas import tpu as pltpu

EMBED = 768
NHEADS = 12
HD = 64
CHUNK = 32
SEQ = 2048
NCHUNK = SEQ // CHUNK
SB = 256
NSB = SEQ // SB
SCALE = 1.0 / float(np.sqrt(HD))
BASE = 10000.0
WALL = 256


def _rope(t, cos_i, sin_i, lane):
    left = jnp.concatenate([t[:, 1:], t[:, :1]], axis=1)
    right = jnp.concatenate([t[:, -1:], t[:, :-1]], axis=1)
    y = jnp.where(lane % 2 == 0, left, right)
    return t * cos_i + y * sin_i


def _pack_kernel(wq_ref, wk_ref, wv_ref, wg_ref, w_ref):
    wq = wq_ref[...]
    wk = wk_ref[...]
    wv = wv_ref[...]
    wg = wg_ref[...]
    pad = jnp.zeros((EMBED, WALL - 3 * HD - 2), jnp.float32)
    for h in range(NHEADS):
        w_ref[h] = jnp.concatenate(
            [wq[:, h * HD:(h + 1) * HD], wk[:, h * HD:(h + 1) * HD],
             wv[:, h * HD:(h + 1) * HD], wg[:, 2 * h:2 * h + 2], pad],
            axis=1)


def _proj_kernel(x_ref, w_ref, cos_ref, sin_ref,
                 q_ref, k_ref, v_ref, kc_ref, g_ref):
    h = pl.program_id(1)
    xb = x_ref[...]
    allp = jnp.dot(xb, w_ref[h], preferred_element_type=jnp.float32)
    q = allp[:, 0:HD]
    k = allp[:, HD:2 * HD]
    v = allp[:, 2 * HD:3 * HD]
    ga = allp[:, 3 * HD:3 * HD + 1]
    gb = allp[:, 3 * HD + 1:3 * HD + 2]
    lane = jax.lax.broadcasted_iota(jnp.int32, (SB, HD), 1)
    kr = _rope(k, cos_ref[...], sin_ref[...], lane)
    q_ref[0] = _rope(q, cos_ref[...], sin_ref[...], lane)
    k_ref[0] = kr
    v_ref[0] = v
    g_ref[0] = jnp.concatenate(
        [jax.nn.sigmoid(ga - gb), jax.nn.sigmoid(gb - ga)], axis=1)
    r = jax.lax.broadcasted_iota(jnp.int32, (SB // CHUNK, SB), 0)
    c = jax.lax.broadcasted_iota(jnp.int32, (SB // CHUNK, SB), 1) // CHUNK
    m8 = jnp.where(r == c, 1.0 / CHUNK, 0.0).astype(jnp.bfloat16)
    hi = kr.astype(jnp.bfloat16)
    lo1 = kr - hi.astype(jnp.float32)
    lo1h = lo1.astype(jnp.bfloat16)
    lo2 = (lo1 - lo1h.astype(jnp.float32)).astype(jnp.bfloat16)
    acc = jnp.dot(m8, hi, preferred_element_type=jnp.float32)
    acc = acc + jnp.dot(m8, lo1h, preferred_element_type=jnp.float32)
    acc = acc + jnp.dot(m8, lo2, preferred_element_type=jnp.float32)
    kc_ref[0] = acc


def _attn_kernel(q_ref, k_ref, v_ref, kc_ref, g_ref, wo_ref, lm_ref,
                 e_ref, out_ref, s_ref, acc_ref, den_ref, m_ref, oi_ref):
    i = pl.program_id(0)
    h = pl.program_id(1)
    qb = q_ref[0]
    kc = kc_ref[h]

    score = jax.lax.dot_general(qb, kc, (((1,), (1,)), ((), ())),
                                preferred_element_type=jnp.float32)
    qpos = jax.lax.broadcasted_iota(jnp.int32, (SB, NCHUNK), 0) + i * SB
    g = jax.lax.broadcasted_iota(jnp.int32, (SB, NCHUNK), 1)
    sentinel = -1e30 * (1.0 + 0.001 * g.astype(jnp.float32))
    sf = jnp.where(qpos < CHUNK * g, score, sentinel)
    m1 = jnp.max(sf, axis=1, keepdims=True)
    i1 = jnp.min(jnp.where(sf == m1, g, NCHUNK), axis=1, keepdims=True)
    sf2 = jnp.where(g == i1, -1e31, sf)
    m2 = jnp.max(sf2, axis=1, keepdims=True)
    i2 = jnp.min(jnp.where(sf2 == m2, g, NCHUNK), axis=1, keepdims=True)
    allow_c = (g == i1) | (g == i2)

    qs = qb * SCALE
    m_ref[...] = jnp.full((SB, 1), -1e30, jnp.float32)
    for j in range(NSB):
        cond = (i <= j) if j else ((i <= j) | (i == NSB - 1))

        @pl.when(cond)
        def _(j=j):
            kt = k_ref[h, j * SB:(j + 1) * SB, :]
            s_j = jax.lax.dot_general(qs, kt, (((1,), (1,)), ((), ())),
                                      preferred_element_type=jnp.float32)
            s_ref[:, j * SB:(j + 1) * SB] = s_j
            m_ref[...] = jnp.maximum(
                m_ref[...], jnp.max(s_j, axis=1, keepdims=True))

        @pl.when(i == j)
        def _(j=j):
            s2 = s_ref[:, j * SB:(j + 1) * SB] + lm_ref[...]
            m2_ = jnp.max(s2, axis=1, keepdims=True)
            p2 = jnp.exp(s2 - m2_)
            vl = v_ref[h, j * SB:(j + 1) * SB, :]
            o_intra = jnp.dot(p2, vl, preferred_element_type=jnp.float32)
            oi_ref[...] = o_intra / jnp.sum(p2, axis=1, keepdims=True)

    mi = m_ref[...]
    bias_c = jnp.where(allow_c, -mi, -1e30)
    acc_ref[...] = jnp.zeros((SB, HD), jnp.float32)
    den_ref[...] = jnp.zeros((SB, 1), jnp.float32)
    for j in range(NSB):
        cond = (i <= j) if j else ((i <= j) | (i == NSB - 1))

        @pl.when(cond)
        def _(j=j):
            e_j = e_ref[:, j * SB:(j + 1) * SB]
            bias_wj = jnp.dot(bias_c, e_j,
                              preferred_element_type=jnp.float32)
            p_j = jnp.exp(s_ref[:, j * SB:(j + 1) * SB] + bias_wj)
            vt = v_ref[h, j * SB:(j + 1) * SB, :]
            acc_ref[...] = acc_ref[...] + jnp.dot(
                p_j, vt, preferred_element_type=jnp.float32)
            den_ref[...] = den_ref[...] + jnp.sum(p_j, axis=1,
                                                  keepdims=True)
    o_inter = acc_ref[...] / den_ref[...]

    g01 = g_ref[0]
    o = g01[:, 0:1] * o_inter + g01[:, 1:2] * oi_ref[...]
    partial = jnp.dot(o, wo_ref[h], preferred_element_type=jnp.float32)

    @pl.when(h == 0)
    def _():
        out_ref[...] = partial

    @pl.when(h != 0)
    def _():
        out_ref[...] = out_ref[...] + partial


def kernel(x, Wq, Wk, Wv, Wg, Wo):
    x2 = x.reshape(SEQ, EMBED)
    w_all = pl.pallas_call(
        _pack_kernel,
        grid=(1,),
        in_specs=[
            pl.BlockSpec((EMBED, EMBED), lambda j: (0, 0)),
            pl.BlockSpec((EMBED, EMBED), lambda j: (0, 0)),
            pl.BlockSpec((EMBED, EMBED), lambda j: (0, 0)),
            pl.BlockSpec((EMBED, 2 * NHEADS), lambda j: (0, 0)),
        ],
        out_specs=pl.BlockSpec((NHEADS, EMBED, WALL), lambda j: (0, 0, 0)),
        out_shape=jax.ShapeDtypeStruct((NHEADS, EMBED, WALL), jnp.float32),
    )(Wq, Wk, Wv, Wg)
    wo_r = Wo.reshape(NHEADS, HD, EMBED)

    qr_ = np.arange(SB)[:, None]
    kr_ = np.arange(SB)[None, :]
    lmask = jnp.asarray(np.where(
        (qr_ // CHUNK == kr_ // CHUNK) & (kr_ <= qr_), 0.0, -1e30),
        dtype=jnp.float32)
    emat = jnp.asarray(
        (np.arange(SEQ)[None, :] // CHUNK == np.arange(NCHUNK)[:, None])
        .astype(np.float32))

    pos = jnp.arange(SEQ, dtype=jnp.float32)
    inv = 1.0 / (BASE ** (jnp.arange(0, HD, 2, dtype=jnp.float32) / HD))
    freqs = pos[:, None] * inv[None, :]
    cos_h = jnp.cos(freqs)
    sin_h = jnp.sin(freqs)
    cos_i = jnp.stack([cos_h, cos_h], axis=-1).reshape(SEQ, HD)
    sin_i = jnp.stack([-sin_h, sin_h], axis=-1).reshape(SEQ, HD)

    q_t, k_t, v_t, kc, g01 = pl.pallas_call(
        _proj_kernel,
        grid=(NSB, NHEADS),
        in_specs=[
            pl.BlockSpec((SB, EMBED), lambda i, h: (i, 0)),
            pl.BlockSpec((NHEADS, EMBED, WALL), lambda i, h: (0, 0, 0)),
            pl.BlockSpec((SB, HD), lambda i, h: (i, 0)),
            pl.BlockSpec((SB, HD), lambda i, h: (i, 0)),
        ],
        out_specs=[
            pl.BlockSpec((1, SB, HD), lambda i, h: (h, i, 0)),
            pl.BlockSpec((1, SB, HD), lambda i, h: (h, i, 0)),
            pl.BlockSpec((1, SB, HD), lambda i, h: (h, i, 0)),
            pl.BlockSpec((1, SB // CHUNK, HD), lambda i, h: (h, i, 0)),
            pl.BlockSpec((1, SB, 2), lambda i, h: (h, i, 0)),
        ],
        out_shape=[
            jax.ShapeDtypeStruct((NHEADS, SEQ, HD), jnp.float32),
            jax.ShapeDtypeStruct((NHEADS, SEQ, HD), jnp.float32),
            jax.ShapeDtypeStruct((NHEADS, SEQ, HD), jnp.float32),
            jax.ShapeDtypeStruct((NHEADS, NCHUNK, HD), jnp.float32),
            jax.ShapeDtypeStruct((NHEADS, SEQ, 2), jnp.float32),
        ],
    )(x2, w_all, cos_i, sin_i)

    out = pl.pallas_call(
        _attn_kernel,
        grid=(NSB, NHEADS),
        in_specs=[
            pl.BlockSpec((1, SB, HD), lambda i, h: (h, i, 0)),
            pl.BlockSpec((NHEADS, SEQ, HD), lambda i, h: (0, 0, 0)),
            pl.BlockSpec((NHEADS, SEQ, HD), lambda i, h: (0, 0, 0)),
            pl.BlockSpec((NHEADS, NCHUNK, HD), lambda i, h: (0, 0, 0)),
            pl.BlockSpec((1, SB, 2), lambda i, h: (h, i, 0)),
            pl.BlockSpec((NHEADS, HD, EMBED), lambda i, h: (0, 0, 0)),
            pl.BlockSpec((SB, SB), lambda i, h: (0, 0)),
            pl.BlockSpec((NCHUNK, SEQ), lambda i, h: (0, 0)),
        ],
        out_specs=pl.BlockSpec((SB, EMBED), lambda i, h: (i, 0)),
        out_shape=jax.ShapeDtypeStruct((SEQ, EMBED), jnp.float32),
        scratch_shapes=[
            pltpu.VMEM((SB, SEQ), jnp.float32),
            pltpu.VMEM((SB, HD), jnp.float32),
            pltpu.VMEM((SB, 1), jnp.float32),
            pltpu.VMEM((SB, 1), jnp.float32),
            pltpu.VMEM((SB, HD), jnp.float32),
        ],
    )(q_t, k_t, v_t, kc, g01, wo_r, lmask, emat)
    return out.reshape(1, SEQ, EMBED)


# speedup vs baseline: 1.6622x; 1.6622x over previous
import numpy as np
import jax
import jax.numpy as jnp
from jax.experimental import pallas as pl
from jax.experimental.pallas import tpu as pltpu

EMBED = 768
NHEADS = 12
HD = 64
CHUNK = 32
SEQ = 2048
NCHUNK = SEQ // CHUNK
SB = 256
NSB = SEQ // SB
SCALE = 1.0 / float(np.sqrt(HD))
BASE = 10000.0
WALL = 256


def _rope(t, cos_i, sin_i, lane):
    left = jnp.concatenate([t[:, 1:], t[:, :1]], axis=1)
    right = jnp.concatenate([t[:, -1:], t[:, :-1]], axis=1)
    y = jnp.where(lane % 2 == 0, left, right)
    return t * cos_i + y * sin_i


def _pack_kernel(wq_ref, wk_ref, wv_ref, wg_ref, w_ref):
    wq = wq_ref[...]
    wk = wk_ref[...]
    wv = wv_ref[...]
    wg = wg_ref[...]
    pad = jnp.zeros((EMBED, WALL - 3 * HD - 2), jnp.float32)
    for h in range(NHEADS):
        w_ref[h] = jnp.concatenate(
            [wq[:, h * HD:(h + 1) * HD], wk[:, h * HD:(h + 1) * HD],
             wv[:, h * HD:(h + 1) * HD], wg[:, 2 * h:2 * h + 2], pad],
            axis=1)


def _proj_kernel(x_ref, w_ref, cos_ref, sin_ref,
                 q_ref, k_ref, v_ref, kc_ref, g_ref):
    h = pl.program_id(1)
    xb = x_ref[...]
    allp = jnp.dot(xb, w_ref[h], preferred_element_type=jnp.float32)
    q = allp[:, 0:HD]
    k = allp[:, HD:2 * HD]
    v = allp[:, 2 * HD:3 * HD]
    ga = allp[:, 3 * HD:3 * HD + 1]
    gb = allp[:, 3 * HD + 1:3 * HD + 2]
    lane = jax.lax.broadcasted_iota(jnp.int32, (SB, HD), 1)
    kr = _rope(k, cos_ref[...], sin_ref[...], lane)
    q_ref[0] = _rope(q, cos_ref[...], sin_ref[...], lane)
    k_ref[0] = kr
    v_ref[0] = v
    g_ref[0] = jnp.concatenate(
        [jax.nn.sigmoid(ga - gb), jax.nn.sigmoid(gb - ga)], axis=1)
    r = jax.lax.broadcasted_iota(jnp.int32, (SB // CHUNK, SB), 0)
    c = jax.lax.broadcasted_iota(jnp.int32, (SB // CHUNK, SB), 1) // CHUNK
    m8 = jnp.where(r == c, 1.0 / CHUNK, 0.0).astype(jnp.bfloat16)
    hi = kr.astype(jnp.bfloat16)
    lo1 = kr - hi.astype(jnp.float32)
    lo1h = lo1.astype(jnp.bfloat16)
    lo2 = (lo1 - lo1h.astype(jnp.float32)).astype(jnp.bfloat16)
    acc = jnp.dot(m8, hi, preferred_element_type=jnp.float32)
    acc = acc + jnp.dot(m8, lo1h, preferred_element_type=jnp.float32)
    acc = acc + jnp.dot(m8, lo2, preferred_element_type=jnp.float32)
    kc_ref[0] = acc


def _attn_kernel(q_ref, k_ref, v_ref, kc_ref, g_ref, wo_ref, lm_ref,
                 e_ref, out_ref):
    i = pl.program_id(0)
    h = pl.program_id(1)
    qb = q_ref[0]
    kk = k_ref[h]
    vv = v_ref[h]
    kc = kc_ref[h]

    score = jax.lax.dot_general(qb, kc, (((1,), (1,)), ((), ())),
                                preferred_element_type=jnp.float32)
    qpos = jax.lax.broadcasted_iota(jnp.int32, (SB, NCHUNK), 0) + i * SB
    g = jax.lax.broadcasted_iota(jnp.int32, (SB, NCHUNK), 1)
    sentinel = -1e30 * (1.0 + 0.001 * g.astype(jnp.float32))
    sf = jnp.where(qpos < CHUNK * g, score, sentinel)
    m1 = jnp.max(sf, axis=1, keepdims=True)
    i1 = jnp.min(jnp.where(sf == m1, g, NCHUNK), axis=1, keepdims=True)
    sf2 = jnp.where(g == i1, -1e31, sf)
    m2 = jnp.max(sf2, axis=1, keepdims=True)
    i2 = jnp.min(jnp.where(sf2 == m2, g, NCHUNK), axis=1, keepdims=True)

    qs = qb * SCALE
    s = jax.lax.dot_general(qs, kk, (((1,), (1,)), ((), ())),
                            preferred_element_type=jnp.float32)
    mi = jnp.max(s, axis=1, keepdims=True)
    bias_c = jnp.where((g == i1) | (g == i2), -mi, -1e30)
    bias_w = jnp.dot(bias_c, e_ref[...], preferred_element_type=jnp.float32)
    p = jnp.exp(s + bias_w)
    o_inter = jnp.dot(p, vv, preferred_element_type=jnp.float32)
    o_inter = o_inter / jnp.sum(p, axis=1, keepdims=True)

    kl = k_ref[h, pl.ds(i * SB, SB), :]
    vl = v_ref[h, pl.ds(i * SB, SB), :]
    s2 = jax.lax.dot_general(qs, kl, (((1,), (1,)), ((), ())),
                             preferred_element_type=jnp.float32)
    s2 = s2 + lm_ref[...]
    m2_ = jnp.max(s2, axis=1, keepdims=True)
    p2 = jnp.exp(s2 - m2_)
    o_intra = jnp.dot(p2, vl, preferred_element_type=jnp.float32)
    o_intra = o_intra / jnp.sum(p2, axis=1, keepdims=True)

    g01 = g_ref[0]
    o = g01[:, 0:1] * o_inter + g01[:, 1:2] * o_intra
    partial = jnp.dot(o, wo_ref[h], preferred_element_type=jnp.float32)

    @pl.when(h == 0)
    def _():
        out_ref[...] = partial

    @pl.when(h != 0)
    def _():
        out_ref[...] = out_ref[...] + partial


def kernel(x, Wq, Wk, Wv, Wg, Wo):
    x2 = x.reshape(SEQ, EMBED)
    w_all = pl.pallas_call(
        _pack_kernel,
        grid=(1,),
        in_specs=[
            pl.BlockSpec((EMBED, EMBED), lambda j: (0, 0)),
            pl.BlockSpec((EMBED, EMBED), lambda j: (0, 0)),
            pl.BlockSpec((EMBED, EMBED), lambda j: (0, 0)),
            pl.BlockSpec((EMBED, 2 * NHEADS), lambda j: (0, 0)),
        ],
        out_specs=pl.BlockSpec((NHEADS, EMBED, WALL), lambda j: (0, 0, 0)),
        out_shape=jax.ShapeDtypeStruct((NHEADS, EMBED, WALL), jnp.float32),
    )(Wq, Wk, Wv, Wg)
    wo_r = Wo.reshape(NHEADS, HD, EMBED)

    qr_ = np.arange(SB)[:, None]
    kr_ = np.arange(SB)[None, :]
    lmask = jnp.asarray(np.where(
        (qr_ // CHUNK == kr_ // CHUNK) & (kr_ <= qr_), 0.0, -1e30),
        dtype=jnp.float32)
    emat = jnp.asarray(
        (np.arange(SEQ)[None, :] // CHUNK == np.arange(NCHUNK)[:, None])
        .astype(np.float32))

    pos = jnp.arange(SEQ, dtype=jnp.float32)
    inv = 1.0 / (BASE ** (jnp.arange(0, HD, 2, dtype=jnp.float32) / HD))
    freqs = pos[:, None] * inv[None, :]
    cos_h = jnp.cos(freqs)
    sin_h = jnp.sin(freqs)
    cos_i = jnp.stack([cos_h, cos_h], axis=-1).reshape(SEQ, HD)
    sin_i = jnp.stack([-sin_h, sin_h], axis=-1).reshape(SEQ, HD)

    q_t, k_t, v_t, kc, g01 = pl.pallas_call(
        _proj_kernel,
        grid=(NSB, NHEADS),
        in_specs=[
            pl.BlockSpec((SB, EMBED), lambda i, h: (i, 0)),
            pl.BlockSpec((NHEADS, EMBED, WALL), lambda i, h: (0, 0, 0)),
            pl.BlockSpec((SB, HD), lambda i, h: (i, 0)),
            pl.BlockSpec((SB, HD), lambda i, h: (i, 0)),
        ],
        out_specs=[
            pl.BlockSpec((1, SB, HD), lambda i, h: (h, i, 0)),
            pl.BlockSpec((1, SB, HD), lambda i, h: (h, i, 0)),
            pl.BlockSpec((1, SB, HD), lambda i, h: (h, i, 0)),
            pl.BlockSpec((1, SB // CHUNK, HD), lambda i, h: (h, i, 0)),
            pl.BlockSpec((1, SB, 2), lambda i, h: (h, i, 0)),
        ],
        out_shape=[
            jax.ShapeDtypeStruct((NHEADS, SEQ, HD), jnp.float32),
            jax.ShapeDtypeStruct((NHEADS, SEQ, HD), jnp.float32),
            jax.ShapeDtypeStruct((NHEADS, SEQ, HD), jnp.float32),
            jax.ShapeDtypeStruct((NHEADS, NCHUNK, HD), jnp.float32),
            jax.ShapeDtypeStruct((NHEADS, SEQ, 2), jnp.float32),
        ],
    )(x2, w_all, cos_i, sin_i)

    out = pl.pallas_call(
        _attn_kernel,
        grid=(NSB, NHEADS),
        in_specs=[
            pl.BlockSpec((1, SB, HD), lambda i, h: (h, i, 0)),
            pl.BlockSpec((NHEADS, SEQ, HD), lambda i, h: (0, 0, 0)),
            pl.BlockSpec((NHEADS, SEQ, HD), lambda i, h: (0, 0, 0)),
            pl.BlockSpec((NHEADS, NCHUNK, HD), lambda i, h: (0, 0, 0)),
            pl.BlockSpec((1, SB, 2), lambda i, h: (h, i, 0)),
            pl.BlockSpec((NHEADS, HD, EMBED), lambda i, h: (0, 0, 0)),
            pl.BlockSpec((SB, SB), lambda i, h: (0, 0)),
            pl.BlockSpec((NCHUNK, SEQ), lambda i, h: (0, 0)),
        ],
        out_specs=pl.BlockSpec((SB, EMBED), lambda i, h: (i, 0)),
        out_shape=jax.ShapeDtypeStruct((SEQ, EMBED), jnp.float32),
    )(q_t, k_t, v_t, kc, g01, wo_r, lmask, emat)
    return out.reshape(1, SEQ, EMBED)


# speedup vs baseline: 1.6746x; 1.0075x over previous
import numpy as np
import jax
import jax.numpy as jnp
from jax.experimental import pallas as pl
from jax.experimental.pallas import tpu as pltpu

EMBED = 768
NHEADS = 12
HD = 64
CHUNK = 32
SEQ = 2048
NCHUNK = SEQ // CHUNK
SB = 256
NSB = SEQ // SB
SCALE = 1.0 / float(np.sqrt(HD))
BASE = 10000.0
WALL = 256


def _rope(t, cos_i, sin_i, lane):
    left = jnp.concatenate([t[:, 1:], t[:, :1]], axis=1)
    right = jnp.concatenate([t[:, -1:], t[:, :-1]], axis=1)
    y = jnp.where(lane % 2 == 0, left, right)
    return t * cos_i + y * sin_i


def _pack_kernel(wq_ref, wk_ref, wv_ref, wg_ref, w_ref):
    wq = wq_ref[...]
    wk = wk_ref[...]
    wv = wv_ref[...]
    wg = wg_ref[...]
    pad = jnp.zeros((EMBED, WALL - 3 * HD - 2), jnp.float32)
    for h in range(NHEADS):
        w_ref[h] = jnp.concatenate(
            [wq[:, h * HD:(h + 1) * HD], wk[:, h * HD:(h + 1) * HD],
             wv[:, h * HD:(h + 1) * HD], wg[:, 2 * h:2 * h + 2], pad],
            axis=1)


def _proj_kernel(x_ref, w_ref, cos_ref, sin_ref,
                 q_ref, k_ref, v_ref, kc_ref, g_ref):
    h = pl.program_id(1)
    xb = x_ref[...]
    allp = jnp.dot(xb, w_ref[h], preferred_element_type=jnp.float32)
    q = allp[:, 0:HD]
    k = allp[:, HD:2 * HD]
    v = allp[:, 2 * HD:3 * HD]
    ga = allp[:, 3 * HD:3 * HD + 1]
    gb = allp[:, 3 * HD + 1:3 * HD + 2]
    lane = jax.lax.broadcasted_iota(jnp.int32, (SB, HD), 1)
    kr = _rope(k, cos_ref[...], sin_ref[...], lane)
    q_ref[0] = _rope(q, cos_ref[...], sin_ref[...], lane).astype(jnp.bfloat16)
    k_ref[0] = kr.astype(jnp.bfloat16)
    v_ref[0] = v.astype(jnp.bfloat16)
    g_ref[0] = jnp.concatenate(
        [jax.nn.sigmoid(ga - gb), jax.nn.sigmoid(gb - ga)], axis=1)
    r = jax.lax.broadcasted_iota(jnp.int32, (SB // CHUNK, SB), 0)
    c = jax.lax.broadcasted_iota(jnp.int32, (SB // CHUNK, SB), 1) // CHUNK
    m8 = jnp.where(r == c, 1.0 / CHUNK, 0.0).astype(jnp.bfloat16)
    hi = kr.astype(jnp.bfloat16)
    lo1 = kr - hi.astype(jnp.float32)
    lo1h = lo1.astype(jnp.bfloat16)
    lo2 = (lo1 - lo1h.astype(jnp.float32)).astype(jnp.bfloat16)
    acc = jnp.dot(m8, hi, preferred_element_type=jnp.float32)
    acc = acc + jnp.dot(m8, lo1h, preferred_element_type=jnp.float32)
    acc = acc + jnp.dot(m8, lo2, preferred_element_type=jnp.float32)
    kc_ref[0] = acc


def _attn_kernel(q_ref, k_ref, v_ref, kc_ref, g_ref, wo_ref, lm_ref,
                 e_ref, out_ref):
    i = pl.program_id(0)
    h = pl.program_id(1)
    qb = q_ref[0]
    kk = k_ref[h]
    vv = v_ref[h]
    kc = kc_ref[h]

    score = jax.lax.dot_general(qb, kc.astype(jnp.bfloat16),
                                (((1,), (1,)), ((), ())),
                                preferred_element_type=jnp.float32)
    qpos = jax.lax.broadcasted_iota(jnp.int32, (SB, NCHUNK), 0) + i * SB
    g = jax.lax.broadcasted_iota(jnp.int32, (SB, NCHUNK), 1)
    sentinel = -1e30 * (1.0 + 0.001 * g.astype(jnp.float32))
    sf = jnp.where(qpos < CHUNK * g, score, sentinel)
    m1 = jnp.max(sf, axis=1, keepdims=True)
    i1 = jnp.min(jnp.where(sf == m1, g, NCHUNK), axis=1, keepdims=True)
    sf2 = jnp.where(g == i1, -1e31, sf)
    m2 = jnp.max(sf2, axis=1, keepdims=True)
    i2 = jnp.min(jnp.where(sf2 == m2, g, NCHUNK), axis=1, keepdims=True)

    qs = qb * jnp.bfloat16(SCALE)
    s = jax.lax.dot_general(qs, kk, (((1,), (1,)), ((), ())),
                            preferred_element_type=jnp.float32)
    mi = jnp.max(s, axis=1, keepdims=True)
    bias_c = jnp.where((g == i1) | (g == i2), -mi, -1e30)
    bias_w = jnp.dot(bias_c, e_ref[...], preferred_element_type=jnp.float32)
    p = jnp.exp(s + bias_w)
    o_inter = jnp.dot(p.astype(jnp.bfloat16), vv,
                      preferred_element_type=jnp.float32)
    o_inter = o_inter / jnp.sum(p, axis=1, keepdims=True)

    kl = k_ref[h, pl.ds(i * SB, SB), :]
    vl = v_ref[h, pl.ds(i * SB, SB), :]
    s2 = jax.lax.dot_general(qs, kl, (((1,), (1,)), ((), ())),
                             preferred_element_type=jnp.float32)
    s2 = s2 + lm_ref[...]
    m2_ = jnp.max(s2, axis=1, keepdims=True)
    p2 = jnp.exp(s2 - m2_)
    o_intra = jnp.dot(p2.astype(jnp.bfloat16), vl,
                       preferred_element_type=jnp.float32)
    o_intra = o_intra / jnp.sum(p2, axis=1, keepdims=True)

    g01 = g_ref[0]
    o = g01[:, 0:1] * o_inter + g01[:, 1:2] * o_intra
    partial = jnp.dot(o, wo_ref[h], preferred_element_type=jnp.float32)

    @pl.when(h == 0)
    def _():
        out_ref[...] = partial

    @pl.when(h != 0)
    def _():
        out_ref[...] = out_ref[...] + partial


def kernel(x, Wq, Wk, Wv, Wg, Wo):
    x2 = x.reshape(SEQ, EMBED)
    w_all = pl.pallas_call(
        _pack_kernel,
        grid=(1,),
        in_specs=[
            pl.BlockSpec((EMBED, EMBED), lambda j: (0, 0)),
            pl.BlockSpec((EMBED, EMBED), lambda j: (0, 0)),
            pl.BlockSpec((EMBED, EMBED), lambda j: (0, 0)),
            pl.BlockSpec((EMBED, 2 * NHEADS), lambda j: (0, 0)),
        ],
        out_specs=pl.BlockSpec((NHEADS, EMBED, WALL), lambda j: (0, 0, 0)),
        out_shape=jax.ShapeDtypeStruct((NHEADS, EMBED, WALL), jnp.float32),
    )(Wq, Wk, Wv, Wg)
    wo_r = Wo.reshape(NHEADS, HD, EMBED)

    qr_ = np.arange(SB)[:, None]
    kr_ = np.arange(SB)[None, :]
    lmask = jnp.asarray(np.where(
        (qr_ // CHUNK == kr_ // CHUNK) & (kr_ <= qr_), 0.0, -1e30),
        dtype=jnp.float32)
    emat = jnp.asarray(
        (np.arange(SEQ)[None, :] // CHUNK == np.arange(NCHUNK)[:, None])
        .astype(np.float32))

    pos = jnp.arange(SEQ, dtype=jnp.float32)
    inv = 1.0 / (BASE ** (jnp.arange(0, HD, 2, dtype=jnp.float32) / HD))
    freqs = pos[:, None] * inv[None, :]
    cos_h = jnp.cos(freqs)
    sin_h = jnp.sin(freqs)
    cos_i = jnp.stack([cos_h, cos_h], axis=-1).reshape(SEQ, HD)
    sin_i = jnp.stack([-sin_h, sin_h], axis=-1).reshape(SEQ, HD)

    q_t, k_t, v_t, kc, g01 = pl.pallas_call(
        _proj_kernel,
        grid=(NSB, NHEADS),
        in_specs=[
            pl.BlockSpec((SB, EMBED), lambda i, h: (i, 0)),
            pl.BlockSpec((NHEADS, EMBED, WALL), lambda i, h: (0, 0, 0)),
            pl.BlockSpec((SB, HD), lambda i, h: (i, 0)),
            pl.BlockSpec((SB, HD), lambda i, h: (i, 0)),
        ],
        out_specs=[
            pl.BlockSpec((1, SB, HD), lambda i, h: (h, i, 0)),
            pl.BlockSpec((1, SB, HD), lambda i, h: (h, i, 0)),
            pl.BlockSpec((1, SB, HD), lambda i, h: (h, i, 0)),
            pl.BlockSpec((1, SB // CHUNK, HD), lambda i, h: (h, i, 0)),
            pl.BlockSpec((1, SB, 2), lambda i, h: (h, i, 0)),
        ],
        out_shape=[
            jax.ShapeDtypeStruct((NHEADS, SEQ, HD), jnp.bfloat16),
            jax.ShapeDtypeStruct((NHEADS, SEQ, HD), jnp.bfloat16),
            jax.ShapeDtypeStruct((NHEADS, SEQ, HD), jnp.bfloat16),
            jax.ShapeDtypeStruct((NHEADS, NCHUNK, HD), jnp.float32),
            jax.ShapeDtypeStruct((NHEADS, SEQ, 2), jnp.float32),
        ],
    )(x2, w_all, cos_i, sin_i)

    out = pl.pallas_call(
        _attn_kernel,
        grid=(NSB, NHEADS),
        in_specs=[
            pl.BlockSpec((1, SB, HD), lambda i, h: (h, i, 0)),
            pl.BlockSpec((NHEADS, SEQ, HD), lambda i, h: (0, 0, 0)),
            pl.BlockSpec((NHEADS, SEQ, HD), lambda i, h: (0, 0, 0)),
            pl.BlockSpec((NHEADS, NCHUNK, HD), lambda i, h: (0, 0, 0)),
            pl.BlockSpec((1, SB, 2), lambda i, h: (h, i, 0)),
            pl.BlockSpec((NHEADS, HD, EMBED), lambda i, h: (0, 0, 0)),
            pl.BlockSpec((SB, SB), lambda i, h: (0, 0)),
            pl.BlockSpec((NCHUNK, SEQ), lambda i, h: (0, 0)),
        ],
        out_specs=pl.BlockSpec((SB, EMBED), lambda i, h: (i, 0)),
        out_shape=jax.ShapeDtypeStruct((SEQ, EMBED), jnp.float32),
    )(q_t, k_t, v_t, kc, g01, wo_r, lmask, emat)
    return out.reshape(1, SEQ, EMBED)


# speedup vs baseline: 1.8230x; 1.0886x over previous
import numpy as np
import jax
import jax.numpy as jnp
from jax.experimental import pallas as pl
from jax.experimental.pallas import tpu as pltpu

EMBED = 768
NHEADS = 12
HD = 64
CHUNK = 32
SEQ = 2048
NCHUNK = SEQ // CHUNK
SB = 256
NSB = SEQ // SB
SBA = 512
NSBA = SEQ // SBA
SCALE = 1.0 / float(np.sqrt(HD))
BASE = 10000.0
WALL = 256


def _rope(t, cos_i, sin_i, lane):
    del lane
    lane = jax.lax.broadcasted_iota(jnp.int32, t.shape, 1)
    left = jnp.concatenate([t[:, 1:], t[:, :1]], axis=1)
    right = jnp.concatenate([t[:, -1:], t[:, :-1]], axis=1)
    y = jnp.where(lane % 2 == 0, left, right)
    return t * cos_i + y * sin_i


def _pack_kernel(wq_ref, wk_ref, wv_ref, wg_ref, w_ref):
    wq = wq_ref[...]
    wk = wk_ref[...]
    wv = wv_ref[...]
    wg = wg_ref[...]
    pad = jnp.zeros((EMBED, WALL - 3 * HD - 2), jnp.float32)
    for h in range(NHEADS):
        w_ref[h] = jnp.concatenate(
            [wq[:, h * HD:(h + 1) * HD], wk[:, h * HD:(h + 1) * HD],
             wv[:, h * HD:(h + 1) * HD], wg[:, 2 * h:2 * h + 2], pad],
            axis=1)


def _proj_kernel(x_ref, w_ref, cos_ref, sin_ref,
                 q_ref, k_ref, v_ref, kc_ref, g_ref):
    h = pl.program_id(1)
    xb = x_ref[...]
    allp = jnp.dot(xb, w_ref[h], preferred_element_type=jnp.float32)
    q = allp[:, 0:HD]
    k = allp[:, HD:2 * HD]
    v = allp[:, 2 * HD:3 * HD]
    ga = allp[:, 3 * HD:3 * HD + 1]
    gb = allp[:, 3 * HD + 1:3 * HD + 2]
    lane = None
    kr = _rope(k, cos_ref[...], sin_ref[...], lane)
    q_ref[0] = _rope(q, cos_ref[...], sin_ref[...], lane).astype(jnp.bfloat16)
    k_ref[0] = kr.astype(jnp.bfloat16)
    v_ref[0] = v.astype(jnp.bfloat16)
    g_ref[0] = jnp.concatenate(
        [jax.nn.sigmoid(ga - gb), jax.nn.sigmoid(gb - ga)], axis=1)
    r = jax.lax.broadcasted_iota(jnp.int32, (SBA // CHUNK, SBA), 0)
    c = jax.lax.broadcasted_iota(jnp.int32, (SBA // CHUNK, SBA), 1) // CHUNK
    m8 = jnp.where(r == c, 1.0 / CHUNK, 0.0).astype(jnp.bfloat16)
    hi = kr.astype(jnp.bfloat16)
    lo1 = kr - hi.astype(jnp.float32)
    lo1h = lo1.astype(jnp.bfloat16)
    lo2 = (lo1 - lo1h.astype(jnp.float32)).astype(jnp.bfloat16)
    acc = jnp.dot(m8, hi, preferred_element_type=jnp.float32)
    acc = acc + jnp.dot(m8, lo1h, preferred_element_type=jnp.float32)
    acc = acc + jnp.dot(m8, lo2, preferred_element_type=jnp.float32)
    kc_ref[0] = acc


def _attn_kernel(q_ref, k_ref, v_ref, kc_ref, g_ref, wo_ref, lm_ref,
                 e_ref, out_ref):
    i = pl.program_id(0)
    h = pl.program_id(1)
    qb = q_ref[0]
    kk = k_ref[h]
    vv = v_ref[h]
    kc = kc_ref[h]

    score = jax.lax.dot_general(qb, kc.astype(jnp.bfloat16),
                                (((1,), (1,)), ((), ())),
                                preferred_element_type=jnp.float32)
    qpos = jax.lax.broadcasted_iota(jnp.int32, (SB, NCHUNK), 0) + i * SB
    g = jax.lax.broadcasted_iota(jnp.int32, (SB, NCHUNK), 1)
    sentinel = -1e30 * (1.0 + 0.001 * g.astype(jnp.float32))
    sf = jnp.where(qpos < CHUNK * g, score, sentinel)
    m1 = jnp.max(sf, axis=1, keepdims=True)
    i1 = jnp.min(jnp.where(sf == m1, g, NCHUNK), axis=1, keepdims=True)
    sf2 = jnp.where(g == i1, -1e31, sf)
    m2 = jnp.max(sf2, axis=1, keepdims=True)
    i2 = jnp.min(jnp.where(sf2 == m2, g, NCHUNK), axis=1, keepdims=True)

    qs = qb * jnp.bfloat16(SCALE)
    s = jax.lax.dot_general(qs, kk, (((1,), (1,)), ((), ())),
                            preferred_element_type=jnp.float32)
    mi = jnp.max(s, axis=1, keepdims=True)
    bias_c = jnp.where((g == i1) | (g == i2), -mi, -1e30)
    bias_w = jnp.dot(bias_c, e_ref[...], preferred_element_type=jnp.float32)
    p = jnp.exp(s + bias_w)
    o_inter = jnp.dot(p.astype(jnp.bfloat16), vv,
                      preferred_element_type=jnp.float32)
    o_inter = o_inter / jnp.sum(p, axis=1, keepdims=True)

    kl = k_ref[h, pl.ds(i * SB, SB), :]
    vl = v_ref[h, pl.ds(i * SB, SB), :]
    s2 = jax.lax.dot_general(qs, kl, (((1,), (1,)), ((), ())),
                             preferred_element_type=jnp.float32)
    s2 = s2 + lm_ref[...]
    m2_ = jnp.max(s2, axis=1, keepdims=True)
    p2 = jnp.exp(s2 - m2_)
    o_intra = jnp.dot(p2.astype(jnp.bfloat16), vl,
                       preferred_element_type=jnp.float32)
    o_intra = o_intra / jnp.sum(p2, axis=1, keepdims=True)

    g01 = g_ref[0]
    o = g01[:, 0:1] * o_inter + g01[:, 1:2] * o_intra
    partial = jnp.dot(o, wo_ref[h], preferred_element_type=jnp.float32)

    @pl.when(h == 0)
    def _():
        out_ref[...] = partial

    @pl.when(h != 0)
    def _():
        out_ref[...] = out_ref[...] + partial


def kernel(x, Wq, Wk, Wv, Wg, Wo):
    x2 = x.reshape(SEQ, EMBED)
    w_all = pl.pallas_call(
        _pack_kernel,
        grid=(1,),
        in_specs=[
            pl.BlockSpec((EMBED, EMBED), lambda j: (0, 0)),
            pl.BlockSpec((EMBED, EMBED), lambda j: (0, 0)),
            pl.BlockSpec((EMBED, EMBED), lambda j: (0, 0)),
            pl.BlockSpec((EMBED, 2 * NHEADS), lambda j: (0, 0)),
        ],
        out_specs=pl.BlockSpec((NHEADS, EMBED, WALL), lambda j: (0, 0, 0)),
        out_shape=jax.ShapeDtypeStruct((NHEADS, EMBED, WALL), jnp.float32),
    )(Wq, Wk, Wv, Wg)
    wo_r = Wo.reshape(NHEADS, HD, EMBED)

    qr_ = np.arange(SB)[:, None]
    kr_ = np.arange(SB)[None, :]
    lmask = jnp.asarray(np.where(
        (qr_ // CHUNK == kr_ // CHUNK) & (kr_ <= qr_), 0.0, -1e30),
        dtype=jnp.float32)
    emat = jnp.asarray(
        (np.arange(SEQ)[None, :] // CHUNK == np.arange(NCHUNK)[:, None])
        .astype(np.float32))

    pos = jnp.arange(SEQ, dtype=jnp.float32)
    inv = 1.0 / (BASE ** (jnp.arange(0, HD, 2, dtype=jnp.float32) / HD))
    freqs = pos[:, None] * inv[None, :]
    cos_h = jnp.cos(freqs)
    sin_h = jnp.sin(freqs)
    cos_i = jnp.stack([cos_h, cos_h], axis=-1).reshape(SEQ, HD)
    sin_i = jnp.stack([-sin_h, sin_h], axis=-1).reshape(SEQ, HD)

    q_t, k_t, v_t, kc, g01 = pl.pallas_call(
        _proj_kernel,
        grid=(NSBA, NHEADS),
        in_specs=[
            pl.BlockSpec((SBA, EMBED), lambda i, h: (i, 0)),
            pl.BlockSpec((NHEADS, EMBED, WALL), lambda i, h: (0, 0, 0)),
            pl.BlockSpec((SBA, HD), lambda i, h: (i, 0)),
            pl.BlockSpec((SBA, HD), lambda i, h: (i, 0)),
        ],
        out_specs=[
            pl.BlockSpec((1, SBA, HD), lambda i, h: (h, i, 0)),
            pl.BlockSpec((1, SBA, HD), lambda i, h: (h, i, 0)),
            pl.BlockSpec((1, SBA, HD), lambda i, h: (h, i, 0)),
            pl.BlockSpec((1, SBA // CHUNK, HD), lambda i, h: (h, i, 0)),
            pl.BlockSpec((1, SBA, 2), lambda i, h: (h, i, 0)),
        ],
        out_shape=[
            jax.ShapeDtypeStruct((NHEADS, SEQ, HD), jnp.bfloat16),
            jax.ShapeDtypeStruct((NHEADS, SEQ, HD), jnp.bfloat16),
            jax.ShapeDtypeStruct((NHEADS, SEQ, HD), jnp.bfloat16),
            jax.ShapeDtypeStruct((NHEADS, NCHUNK, HD), jnp.float32),
            jax.ShapeDtypeStruct((NHEADS, SEQ, 2), jnp.float32),
        ],
    )(x2, w_all, cos_i, sin_i)

    out = pl.pallas_call(
        _attn_kernel,
        grid=(NSB, NHEADS),
        in_specs=[
            pl.BlockSpec((1, SB, HD), lambda i, h: (h, i, 0)),
            pl.BlockSpec((NHEADS, SEQ, HD), lambda i, h: (0, 0, 0)),
            pl.BlockSpec((NHEADS, SEQ, HD), lambda i, h: (0, 0, 0)),
            pl.BlockSpec((NHEADS, NCHUNK, HD), lambda i, h: (0, 0, 0)),
            pl.BlockSpec((1, SB, 2), lambda i, h: (h, i, 0)),
            pl.BlockSpec((NHEADS, HD, EMBED), lambda i, h: (0, 0, 0)),
            pl.BlockSpec((SB, SB), lambda i, h: (0, 0)),
            pl.BlockSpec((NCHUNK, SEQ), lambda i, h: (0, 0)),
        ],
        out_specs=pl.BlockSpec((SB, EMBED), lambda i, h: (i, 0)),
        out_shape=jax.ShapeDtypeStruct((SEQ, EMBED), jnp.float32),
    )(q_t, k_t, v_t, kc, g01, wo_r, lmask, emat)
    return out.reshape(1, SEQ, EMBED)


# speedup vs baseline: 2.1213x; 1.1636x over previous
import numpy as np
import jax
import jax.numpy as jnp
from jax.experimental import pallas as pl
from jax.experimental.pallas import tpu as pltpu

EMBED = 768
NHEADS = 12
HD = 64
CHUNK = 32
SEQ = 2048
NCHUNK = SEQ // CHUNK
SB = 512
NSB = SEQ // SB
SBA = 512
NSBA = SEQ // SBA
SCALE = 1.0 / float(np.sqrt(HD))
BASE = 10000.0
WALL = 256


def _rope(t, cos_i, sin_i, lane):
    del lane
    lane = jax.lax.broadcasted_iota(jnp.int32, t.shape, 1)
    left = jnp.concatenate([t[:, 1:], t[:, :1]], axis=1)
    right = jnp.concatenate([t[:, -1:], t[:, :-1]], axis=1)
    y = jnp.where(lane % 2 == 0, left, right)
    return t * cos_i + y * sin_i


def _pack_kernel(wq_ref, wk_ref, wv_ref, wg_ref, w_ref):
    wq = wq_ref[...]
    wk = wk_ref[...]
    wv = wv_ref[...]
    wg = wg_ref[...]
    pad = jnp.zeros((EMBED, WALL - 3 * HD - 2), jnp.float32)
    for h in range(NHEADS):
        w_ref[h] = jnp.concatenate(
            [wq[:, h * HD:(h + 1) * HD], wk[:, h * HD:(h + 1) * HD],
             wv[:, h * HD:(h + 1) * HD], wg[:, 2 * h:2 * h + 2], pad],
            axis=1)


def _proj_kernel(x_ref, w_ref, cos_ref, sin_ref,
                 q_ref, k_ref, v_ref, kc_ref, g_ref):
    h = pl.program_id(1)
    xb = x_ref[...]
    allp = jnp.dot(xb, w_ref[h], preferred_element_type=jnp.float32)
    q = allp[:, 0:HD]
    k = allp[:, HD:2 * HD]
    v = allp[:, 2 * HD:3 * HD]
    ga = allp[:, 3 * HD:3 * HD + 1]
    gb = allp[:, 3 * HD + 1:3 * HD + 2]
    lane = None
    kr = _rope(k, cos_ref[...], sin_ref[...], lane)
    q_ref[0] = _rope(q, cos_ref[...], sin_ref[...], lane).astype(jnp.bfloat16)
    k_ref[0] = kr.astype(jnp.bfloat16)
    v_ref[0] = v.astype(jnp.bfloat16)
    g_ref[0] = jnp.concatenate(
        [jax.nn.sigmoid(ga - gb), jax.nn.sigmoid(gb - ga)], axis=1)
    r = jax.lax.broadcasted_iota(jnp.int32, (SBA // CHUNK, SBA), 0)
    c = jax.lax.broadcasted_iota(jnp.int32, (SBA // CHUNK, SBA), 1) // CHUNK
    m8 = jnp.where(r == c, 1.0 / CHUNK, 0.0).astype(jnp.bfloat16)
    hi = kr.astype(jnp.bfloat16)
    lo1 = kr - hi.astype(jnp.float32)
    lo1h = lo1.astype(jnp.bfloat16)
    lo2 = (lo1 - lo1h.astype(jnp.float32)).astype(jnp.bfloat16)
    acc = jnp.dot(m8, hi, preferred_element_type=jnp.float32)
    acc = acc + jnp.dot(m8, lo1h, preferred_element_type=jnp.float32)
    acc = acc + jnp.dot(m8, lo2, preferred_element_type=jnp.float32)
    kc_ref[0] = acc


def _attn_kernel(q_ref, k_ref, v_ref, kc_ref, g_ref, wo_ref, lm_ref,
                 e_ref, out_ref):
    i = pl.program_id(0)
    h = pl.program_id(1)
    qb = q_ref[0]
    kk = k_ref[h]
    vv = v_ref[h]
    kc = kc_ref[h]

    score = jax.lax.dot_general(qb, kc.astype(jnp.bfloat16),
                                (((1,), (1,)), ((), ())),
                                preferred_element_type=jnp.float32)
    qpos = jax.lax.broadcasted_iota(jnp.int32, (SB, NCHUNK), 0) + i * SB
    g = jax.lax.broadcasted_iota(jnp.int32, (SB, NCHUNK), 1)
    sentinel = -1e30 * (1.0 + 0.001 * g.astype(jnp.float32))
    sf = jnp.where(qpos < CHUNK * g, score, sentinel)
    m1 = jnp.max(sf, axis=1, keepdims=True)
    i1 = jnp.min(jnp.where(sf == m1, g, NCHUNK), axis=1, keepdims=True)
    sf2 = jnp.where(g == i1, -1e31, sf)
    m2 = jnp.max(sf2, axis=1, keepdims=True)
    i2 = jnp.min(jnp.where(sf2 == m2, g, NCHUNK), axis=1, keepdims=True)

    qs = qb * jnp.bfloat16(SCALE)
    s = jax.lax.dot_general(qs, kk, (((1,), (1,)), ((), ())),
                            preferred_element_type=jnp.float32)
    mi = jnp.max(s, axis=1, keepdims=True)
    bias_c = jnp.where((g == i1) | (g == i2), -mi, -1e30)
    bias_w = jnp.dot(bias_c, e_ref[...], preferred_element_type=jnp.float32)
    p = jnp.exp(s + bias_w)
    o_inter = jnp.dot(p.astype(jnp.bfloat16), vv,
                      preferred_element_type=jnp.float32)
    o_inter = o_inter / jnp.sum(p, axis=1, keepdims=True)

    kl = k_ref[h, pl.ds(i * SB, SB), :]
    vl = v_ref[h, pl.ds(i * SB, SB), :]
    s2 = jax.lax.dot_general(qs, kl, (((1,), (1,)), ((), ())),
                             preferred_element_type=jnp.float32)
    s2 = s2 + lm_ref[...]
    m2_ = jnp.max(s2, axis=1, keepdims=True)
    p2 = jnp.exp(s2 - m2_)
    o_intra = jnp.dot(p2.astype(jnp.bfloat16), vl,
                       preferred_element_type=jnp.float32)
    o_intra = o_intra / jnp.sum(p2, axis=1, keepdims=True)

    g01 = g_ref[0]
    o = g01[:, 0:1] * o_inter + g01[:, 1:2] * o_intra
    partial = jnp.dot(o, wo_ref[h], preferred_element_type=jnp.float32)

    @pl.when(h == 0)
    def _():
        out_ref[...] = partial

    @pl.when(h != 0)
    def _():
        out_ref[...] = out_ref[...] + partial


def kernel(x, Wq, Wk, Wv, Wg, Wo):
    x2 = x.reshape(SEQ, EMBED)
    w_all = pl.pallas_call(
        _pack_kernel,
        grid=(1,),
        in_specs=[
            pl.BlockSpec((EMBED, EMBED), lambda j: (0, 0)),
            pl.BlockSpec((EMBED, EMBED), lambda j: (0, 0)),
            pl.BlockSpec((EMBED, EMBED), lambda j: (0, 0)),
            pl.BlockSpec((EMBED, 2 * NHEADS), lambda j: (0, 0)),
        ],
        out_specs=pl.BlockSpec((NHEADS, EMBED, WALL), lambda j: (0, 0, 0)),
        out_shape=jax.ShapeDtypeStruct((NHEADS, EMBED, WALL), jnp.float32),
    )(Wq, Wk, Wv, Wg)
    wo_r = Wo.reshape(NHEADS, HD, EMBED)

    qr_ = np.arange(SB)[:, None]
    kr_ = np.arange(SB)[None, :]
    lmask = jnp.asarray(np.where(
        (qr_ // CHUNK == kr_ // CHUNK) & (kr_ <= qr_), 0.0, -1e30),
        dtype=jnp.float32)
    emat = jnp.asarray(
        (np.arange(SEQ)[None, :] // CHUNK == np.arange(NCHUNK)[:, None])
        .astype(np.float32))

    pos = jnp.arange(SEQ, dtype=jnp.float32)
    inv = 1.0 / (BASE ** (jnp.arange(0, HD, 2, dtype=jnp.float32) / HD))
    freqs = pos[:, None] * inv[None, :]
    cos_h = jnp.cos(freqs)
    sin_h = jnp.sin(freqs)
    cos_i = jnp.stack([cos_h, cos_h], axis=-1).reshape(SEQ, HD)
    sin_i = jnp.stack([-sin_h, sin_h], axis=-1).reshape(SEQ, HD)

    q_t, k_t, v_t, kc, g01 = pl.pallas_call(
        _proj_kernel,
        grid=(NSBA, NHEADS),
        in_specs=[
            pl.BlockSpec((SBA, EMBED), lambda i, h: (i, 0)),
            pl.BlockSpec((NHEADS, EMBED, WALL), lambda i, h: (0, 0, 0)),
            pl.BlockSpec((SBA, HD), lambda i, h: (i, 0)),
            pl.BlockSpec((SBA, HD), lambda i, h: (i, 0)),
        ],
        out_specs=[
            pl.BlockSpec((1, SBA, HD), lambda i, h: (h, i, 0)),
            pl.BlockSpec((1, SBA, HD), lambda i, h: (h, i, 0)),
            pl.BlockSpec((1, SBA, HD), lambda i, h: (h, i, 0)),
            pl.BlockSpec((1, SBA // CHUNK, HD), lambda i, h: (h, i, 0)),
            pl.BlockSpec((1, SBA, 2), lambda i, h: (h, i, 0)),
        ],
        out_shape=[
            jax.ShapeDtypeStruct((NHEADS, SEQ, HD), jnp.bfloat16),
            jax.ShapeDtypeStruct((NHEADS, SEQ, HD), jnp.bfloat16),
            jax.ShapeDtypeStruct((NHEADS, SEQ, HD), jnp.bfloat16),
            jax.ShapeDtypeStruct((NHEADS, NCHUNK, HD), jnp.float32),
            jax.ShapeDtypeStruct((NHEADS, SEQ, 2), jnp.float32),
        ],
    )(x2, w_all, cos_i, sin_i)

    out = pl.pallas_call(
        _attn_kernel,
        grid=(NSB, NHEADS),
        in_specs=[
            pl.BlockSpec((1, SB, HD), lambda i, h: (h, i, 0)),
            pl.BlockSpec((NHEADS, SEQ, HD), lambda i, h: (0, 0, 0)),
            pl.BlockSpec((NHEADS, SEQ, HD), lambda i, h: (0, 0, 0)),
            pl.BlockSpec((NHEADS, NCHUNK, HD), lambda i, h: (0, 0, 0)),
            pl.BlockSpec((1, SB, 2), lambda i, h: (h, i, 0)),
            pl.BlockSpec((NHEADS, HD, EMBED), lambda i, h: (0, 0, 0)),
            pl.BlockSpec((SB, SB), lambda i, h: (0, 0)),
            pl.BlockSpec((NCHUNK, SEQ), lambda i, h: (0, 0)),
        ],
        out_specs=pl.BlockSpec((SB, EMBED), lambda i, h: (i, 0)),
        out_shape=jax.ShapeDtypeStruct((SEQ, EMBED), jnp.float32),
    )(q_t, k_t, v_t, kc, g01, wo_r, lmask, emat)
    return out.reshape(1, SEQ, EMBED)


# speedup vs baseline: 2.2168x; 1.0450x over previous
import numpy as np
import jax
import jax.numpy as jnp
from jax.experimental import pallas as pl
from jax.experimental.pallas import tpu as pltpu

EMBED = 768
NHEADS = 12
HD = 64
CHUNK = 32
SEQ = 2048
NCHUNK = SEQ // CHUNK
SB = 1024
NSB = SEQ // SB
SBA = 512
NSBA = SEQ // SBA
SCALE = 1.0 / float(np.sqrt(HD))
BASE = 10000.0
WALL = 256


def _rope(t, cos_i, sin_i, lane):
    del lane
    lane = jax.lax.broadcasted_iota(jnp.int32, t.shape, 1)
    left = jnp.concatenate([t[:, 1:], t[:, :1]], axis=1)
    right = jnp.concatenate([t[:, -1:], t[:, :-1]], axis=1)
    y = jnp.where(lane % 2 == 0, left, right)
    return t * cos_i + y * sin_i


def _pack_kernel(wq_ref, wk_ref, wv_ref, wg_ref, w_ref):
    wq = wq_ref[...]
    wk = wk_ref[...]
    wv = wv_ref[...]
    wg = wg_ref[...]
    pad = jnp.zeros((EMBED, WALL - 3 * HD - 2), jnp.float32)
    for h in range(NHEADS):
        w_ref[h] = jnp.concatenate(
            [wq[:, h * HD:(h + 1) * HD], wk[:, h * HD:(h + 1) * HD],
             wv[:, h * HD:(h + 1) * HD], wg[:, 2 * h:2 * h + 2], pad],
            axis=1)


def _proj_kernel(x_ref, w_ref, cos_ref, sin_ref,
                 q_ref, k_ref, v_ref, kc_ref, g_ref):
    h = pl.program_id(1)
    xb = x_ref[...]
    allp = jnp.dot(xb, w_ref[h], preferred_element_type=jnp.float32)
    q = allp[:, 0:HD]
    k = allp[:, HD:2 * HD]
    v = allp[:, 2 * HD:3 * HD]
    ga = allp[:, 3 * HD:3 * HD + 1]
    gb = allp[:, 3 * HD + 1:3 * HD + 2]
    lane = None
    kr = _rope(k, cos_ref[...], sin_ref[...], lane)
    q_ref[0] = _rope(q, cos_ref[...], sin_ref[...], lane).astype(jnp.bfloat16)
    k_ref[0] = kr.astype(jnp.bfloat16)
    v_ref[0] = v.astype(jnp.bfloat16)
    g_ref[0] = jnp.concatenate(
        [jax.nn.sigmoid(ga - gb), jax.nn.sigmoid(gb - ga)], axis=1)
    r = jax.lax.broadcasted_iota(jnp.int32, (SBA // CHUNK, SBA), 0)
    c = jax.lax.broadcasted_iota(jnp.int32, (SBA // CHUNK, SBA), 1) // CHUNK
    m8 = jnp.where(r == c, 1.0 / CHUNK, 0.0).astype(jnp.bfloat16)
    hi = kr.astype(jnp.bfloat16)
    lo1 = kr - hi.astype(jnp.float32)
    lo1h = lo1.astype(jnp.bfloat16)
    lo2 = (lo1 - lo1h.astype(jnp.float32)).astype(jnp.bfloat16)
    acc = jnp.dot(m8, hi, preferred_element_type=jnp.float32)
    acc = acc + jnp.dot(m8, lo1h, preferred_element_type=jnp.float32)
    acc = acc + jnp.dot(m8, lo2, preferred_element_type=jnp.float32)
    kc_ref[0] = acc


def _attn_kernel(q_ref, k_ref, v_ref, kc_ref, g_ref, wo_ref, lm_ref,
                 e_ref, out_ref):
    i = pl.program_id(0)
    h = pl.program_id(1)
    qb = q_ref[0]
    kk = k_ref[h]
    vv = v_ref[h]
    kc = kc_ref[h]

    score = jax.lax.dot_general(qb, kc.astype(jnp.bfloat16),
                                (((1,), (1,)), ((), ())),
                                preferred_element_type=jnp.float32)
    qpos = jax.lax.broadcasted_iota(jnp.int32, (SB, NCHUNK), 0) + i * SB
    g = jax.lax.broadcasted_iota(jnp.int32, (SB, NCHUNK), 1)
    sentinel = -1e30 * (1.0 + 0.001 * g.astype(jnp.float32))
    sf = jnp.where(qpos < CHUNK * g, score, sentinel)
    m1 = jnp.max(sf, axis=1, keepdims=True)
    i1 = jnp.min(jnp.where(sf == m1, g, NCHUNK), axis=1, keepdims=True)
    sf2 = jnp.where(g == i1, -1e31, sf)
    m2 = jnp.max(sf2, axis=1, keepdims=True)
    i2 = jnp.min(jnp.where(sf2 == m2, g, NCHUNK), axis=1, keepdims=True)

    qs = qb * jnp.bfloat16(SCALE)
    s = jax.lax.dot_general(qs, kk, (((1,), (1,)), ((), ())),
                            preferred_element_type=jnp.float32)
    mi = jnp.max(s, axis=1, keepdims=True)
    bias_c = jnp.where((g == i1) | (g == i2), -mi, -1e30)
    bias_w = jnp.dot(bias_c, e_ref[...], preferred_element_type=jnp.float32)
    p = jnp.exp(s + bias_w)
    o_inter = jnp.dot(p.astype(jnp.bfloat16), vv,
                      preferred_element_type=jnp.float32)
    o_inter = o_inter / jnp.sum(p, axis=1, keepdims=True)

    kl = k_ref[h, pl.ds(i * SB, SB), :]
    vl = v_ref[h, pl.ds(i * SB, SB), :]
    s2 = jax.lax.dot_general(qs, kl, (((1,), (1,)), ((), ())),
                             preferred_element_type=jnp.float32)
    s2 = s2 + lm_ref[...]
    m2_ = jnp.max(s2, axis=1, keepdims=True)
    p2 = jnp.exp(s2 - m2_)
    o_intra = jnp.dot(p2.astype(jnp.bfloat16), vl,
                       preferred_element_type=jnp.float32)
    o_intra = o_intra / jnp.sum(p2, axis=1, keepdims=True)

    g01 = g_ref[0]
    o = g01[:, 0:1] * o_inter + g01[:, 1:2] * o_intra
    partial = jnp.dot(o, wo_ref[h], preferred_element_type=jnp.float32)

    @pl.when(h == 0)
    def _():
        out_ref[...] = partial

    @pl.when(h != 0)
    def _():
        out_ref[...] = out_ref[...] + partial


def kernel(x, Wq, Wk, Wv, Wg, Wo):
    x2 = x.reshape(SEQ, EMBED)
    w_all = pl.pallas_call(
        _pack_kernel,
        grid=(1,),
        in_specs=[
            pl.BlockSpec((EMBED, EMBED), lambda j: (0, 0)),
            pl.BlockSpec((EMBED, EMBED), lambda j: (0, 0)),
            pl.BlockSpec((EMBED, EMBED), lambda j: (0, 0)),
            pl.BlockSpec((EMBED, 2 * NHEADS), lambda j: (0, 0)),
        ],
        out_specs=pl.BlockSpec((NHEADS, EMBED, WALL), lambda j: (0, 0, 0)),
        out_shape=jax.ShapeDtypeStruct((NHEADS, EMBED, WALL), jnp.float32),
    )(Wq, Wk, Wv, Wg)
    wo_r = Wo.reshape(NHEADS, HD, EMBED)

    qr_ = np.arange(SB)[:, None]
    kr_ = np.arange(SB)[None, :]
    lmask = jnp.asarray(np.where(
        (qr_ // CHUNK == kr_ // CHUNK) & (kr_ <= qr_), 0.0, -1e30),
        dtype=jnp.float32)
    emat = jnp.asarray(
        (np.arange(SEQ)[None, :] // CHUNK == np.arange(NCHUNK)[:, None])
        .astype(np.float32))

    pos = jnp.arange(SEQ, dtype=jnp.float32)
    inv = 1.0 / (BASE ** (jnp.arange(0, HD, 2, dtype=jnp.float32) / HD))
    freqs = pos[:, None] * inv[None, :]
    cos_h = jnp.cos(freqs)
    sin_h = jnp.sin(freqs)
    cos_i = jnp.stack([cos_h, cos_h], axis=-1).reshape(SEQ, HD)
    sin_i = jnp.stack([-sin_h, sin_h], axis=-1).reshape(SEQ, HD)

    q_t, k_t, v_t, kc, g01 = pl.pallas_call(
        _proj_kernel,
        grid=(NSBA, NHEADS),
        in_specs=[
            pl.BlockSpec((SBA, EMBED), lambda i, h: (i, 0)),
            pl.BlockSpec((NHEADS, EMBED, WALL), lambda i, h: (0, 0, 0)),
            pl.BlockSpec((SBA, HD), lambda i, h: (i, 0)),
            pl.BlockSpec((SBA, HD), lambda i, h: (i, 0)),
        ],
        out_specs=[
            pl.BlockSpec((1, SBA, HD), lambda i, h: (h, i, 0)),
            pl.BlockSpec((1, SBA, HD), lambda i, h: (h, i, 0)),
            pl.BlockSpec((1, SBA, HD), lambda i, h: (h, i, 0)),
            pl.BlockSpec((1, SBA // CHUNK, HD), lambda i, h: (h, i, 0)),
            pl.BlockSpec((1, SBA, 2), lambda i, h: (h, i, 0)),
        ],
        out_shape=[
            jax.ShapeDtypeStruct((NHEADS, SEQ, HD), jnp.bfloat16),
            jax.ShapeDtypeStruct((NHEADS, SEQ, HD), jnp.bfloat16),
            jax.ShapeDtypeStruct((NHEADS, SEQ, HD), jnp.bfloat16),
            jax.ShapeDtypeStruct((NHEADS, NCHUNK, HD), jnp.float32),
            jax.ShapeDtypeStruct((NHEADS, SEQ, 2), jnp.float32),
        ],
    )(x2, w_all, cos_i, sin_i)

    out = pl.pallas_call(
        _attn_kernel,
        grid=(NSB, NHEADS),
        in_specs=[
            pl.BlockSpec((1, SB, HD), lambda i, h: (h, i, 0)),
            pl.BlockSpec((NHEADS, SEQ, HD), lambda i, h: (0, 0, 0)),
            pl.BlockSpec((NHEADS, SEQ, HD), lambda i, h: (0, 0, 0)),
            pl.BlockSpec((NHEADS, NCHUNK, HD), lambda i, h: (0, 0, 0)),
            pl.BlockSpec((1, SB, 2), lambda i, h: (h, i, 0)),
            pl.BlockSpec((NHEADS, HD, EMBED), lambda i, h: (0, 0, 0)),
            pl.BlockSpec((SB, SB), lambda i, h: (0, 0)),
            pl.BlockSpec((NCHUNK, SEQ), lambda i, h: (0, 0)),
        ],
        out_specs=pl.BlockSpec((SB, EMBED), lambda i, h: (i, 0)),
        out_shape=jax.ShapeDtypeStruct((SEQ, EMBED), jnp.float32),
    )(q_t, k_t, v_t, kc, g01, wo_r, lmask, emat)
    return out.reshape(1, SEQ, EMBED)
